# baseline (device time: 21205 ns/iter reference)
import jax
import jax.numpy as jnp
from jax import lax
from jax.experimental import pallas as pl
from jax.experimental.pallas import tpu as pltpu

CK = 128


def kernel(x):
    m, n = x.shape
    q = m // 4
    KQ = q // CK
    KB = KQ // 2
    KY = KQ + KB

    def body(x_ref, out_ref, vx, ins, ys, yr, xsA, xrA, zsA, zrA,
             xsB, xrB, zsB, zrB):
        my_x = lax.axis_index("x")
        my_y = lax.axis_index("y")
        my_z = lax.axis_index("z")
        qz = lax.rem(my_z, 2)
        y_nbr = (my_x, 1 - my_y, my_z)
        x_nbr = (1 - my_x, my_y, my_z)
        z_nbr = (my_x, my_y, my_z + 1 - 2 * qz)

        barrier = pltpu.get_barrier_semaphore()
        for nbr in (y_nbr, x_nbr, z_nbr):
            pl.semaphore_signal(
                barrier, inc=1, device_id=nbr,
                device_id_type=pl.DeviceIdType.MESH,
            )

        mine = my_y * m
        other = (1 - my_y) * m
        Qd = my_x * 2 * q + qz * q
        Qx = (1 - my_x) * 2 * q + qz * q
        Qz = my_x * 2 * q + (1 - qz) * q
        Qg = (1 - my_x) * 2 * q + (1 - qz) * q

        def cast(off, rows):
            out_ref[pl.ds(mine + off, rows), :] = vx[
                pl.ds(off, rows), :
            ].astype(jnp.bfloat16)

        def rc(off, ssem, rsem, dev):
            return pltpu.make_async_remote_copy(
                src_ref=out_ref.at[pl.ds(off, CK)],
                dst_ref=out_ref.at[pl.ds(off, CK)],
                send_sem=ssem,
                recv_sem=rsem,
                device_id=dev,
                device_id_type=pl.DeviceIdType.MESH,
            )

        def rwait(rsem):
            d = pltpu.make_async_remote_copy(
                src_ref=out_ref.at[pl.ds(other, CK)],
                dst_ref=out_ref.at[pl.ds(other, CK)],
                send_sem=ys.at[0],
                recv_sem=rsem,
                device_id=y_nbr,
                device_id_type=pl.DeviceIdType.MESH,
            )
            d.wait_recv()

        y_offs = [Qd + j * CK for j in range(KQ)] + [Qg + i * CK for i in range(KB)]
        slab_offs = [(Qx, q), (Qz, q), (Qg + KB * CK, q - KB * CK)]
        in_dmas = []
        for k, off in enumerate(y_offs):
            c = pltpu.make_async_copy(
                x_ref.at[pl.ds(off, CK)], vx.at[pl.ds(off, CK)], ins.at[k]
            )
            c.start()
            in_dmas.append(c)
        slab_dmas = []
        for k, (off, rows) in enumerate(slab_offs):
            c = pltpu.make_async_copy(
                x_ref.at[pl.ds(off, rows)], vx.at[pl.ds(off, rows)],
                ins.at[KY + k],
            )
            c.start()
            slab_dmas.append(c)

        for k, off in enumerate(y_offs):
            in_dmas[k].wait()
            cast(off, CK)

        pl.semaphore_wait(barrier, 3)

        y_rd = []
        for k, off in enumerate(y_offs):
            r = rc(mine + off, ys.at[k], yr.at[k], y_nbr)
            r.start()
            y_rd.append(r)

        for k, (off, rows) in enumerate(slab_offs):
            slab_dmas[k].wait()
            cast(off, rows)

        xA, zA = [], []
        for j in range(KQ):
            y_rd[j].wait_recv()
            o = other + Qd + j * CK
            r1 = rc(o, xsA.at[j], xrA.at[j], x_nbr)
            r1.start()
            xA.append(r1)
            r2 = rc(o, zsA.at[j], zrA.at[j], z_nbr)
            r2.start()
            zA.append(r2)

        rwait(zrA.at[2])
        xB = rc(other + Qz + 2 * CK, xsB.at[0], xrB.at[0], x_nbr)
        xB.start()
        rwait(xrA.at[3])
        zB = rc(other + Qx + 3 * CK, zsB.at[0], zrB.at[0], z_nbr)
        zB.start()

        for k in range(KQ, KY):
            y_rd[k].wait_recv()
        for j in (0, 1, 2):
            rwait(xrA.at[j])
        for j in (0, 1, 3):
            rwait(zrA.at[j])
        rwait(xrB.at[0])
        rwait(zrB.at[0])

        for k in range(KY):
            y_rd[k].wait_send()
        for j in range(KQ):
            xA[j].wait_send()
            zA[j].wait_send()
        xB.wait_send()
        zB.wait_send()

    return pl.pallas_call(
        body,
        out_shape=jax.ShapeDtypeStruct((2 * m, n), jnp.bfloat16),
        in_specs=[pl.BlockSpec(memory_space=pl.ANY)],
        out_specs=pl.BlockSpec(memory_space=pltpu.VMEM),
        scratch_shapes=[
            pltpu.VMEM((m, n), x.dtype),
            pltpu.SemaphoreType.DMA((KY + 3,)),
            pltpu.SemaphoreType.DMA((KY,)),
            pltpu.SemaphoreType.DMA((KY,)),
            pltpu.SemaphoreType.DMA((KQ,)),
            pltpu.SemaphoreType.DMA((KQ,)),
            pltpu.SemaphoreType.DMA((KQ,)),
            pltpu.SemaphoreType.DMA((KQ,)),
            pltpu.SemaphoreType.DMA((1,)),
            pltpu.SemaphoreType.DMA((1,)),
            pltpu.SemaphoreType.DMA((1,)),
            pltpu.SemaphoreType.DMA((1,)),
        ],
        compiler_params=pltpu.CompilerParams(collective_id=0),
    )(x)


# device time: 20763 ns/iter; 1.0213x vs baseline; 1.0213x over previous
import jax
import jax.numpy as jnp
from jax import lax
from jax.experimental import pallas as pl
from jax.experimental.pallas import tpu as pltpu

CK = 128


def kernel(x):
    m, n = x.shape
    q = m // 4
    KQ = q // CK
    KB = KQ // 2
    KY = KQ + KB

    def body(x_ref, out_ref, ys, yr, xsA, xrA, zsA, zrA, xsB, xrB, zsB, zrB):
        my_x = lax.axis_index("x")
        my_y = lax.axis_index("y")
        my_z = lax.axis_index("z")
        qz = lax.rem(my_z, 2)
        y_nbr = (my_x, 1 - my_y, my_z)
        x_nbr = (1 - my_x, my_y, my_z)
        z_nbr = (my_x, my_y, my_z + 1 - 2 * qz)

        barrier = pltpu.get_barrier_semaphore()
        for nbr in (y_nbr, x_nbr, z_nbr):
            pl.semaphore_signal(
                barrier, inc=1, device_id=nbr,
                device_id_type=pl.DeviceIdType.MESH,
            )

        mine = my_y * m
        other = (1 - my_y) * m
        Qd = my_x * 2 * q + qz * q
        Qx = (1 - my_x) * 2 * q + qz * q
        Qz = my_x * 2 * q + (1 - qz) * q
        Qg = (1 - my_x) * 2 * q + (1 - qz) * q

        def cast(off, rows):
            out_ref[pl.ds(mine + off, rows), :] = x_ref[
                pl.ds(off, rows), :
            ].astype(jnp.bfloat16)

        def rc(off, ssem, rsem, dev):
            return pltpu.make_async_remote_copy(
                src_ref=out_ref.at[pl.ds(off, CK)],
                dst_ref=out_ref.at[pl.ds(off, CK)],
                send_sem=ssem,
                recv_sem=rsem,
                device_id=dev,
                device_id_type=pl.DeviceIdType.MESH,
            )

        def rwait(rsem):
            d = pltpu.make_async_remote_copy(
                src_ref=out_ref.at[pl.ds(other, CK)],
                dst_ref=out_ref.at[pl.ds(other, CK)],
                send_sem=ys.at[0],
                recv_sem=rsem,
                device_id=y_nbr,
                device_id_type=pl.DeviceIdType.MESH,
            )
            d.wait_recv()

        y_offs = [Qd + j * CK for j in range(KQ)] + [Qg + i * CK for i in range(KB)]
        for off in y_offs:
            cast(off, CK)

        pl.semaphore_wait(barrier, 3)

        y_rd = []
        for k, off in enumerate(y_offs):
            r = rc(mine + off, ys.at[k], yr.at[k], y_nbr)
            r.start()
            y_rd.append(r)

        cast(Qx, q)
        cast(Qz, q)
        cast(Qg + KB * CK, q - KB * CK)

        xA, zA = [], []
        for j in range(KQ):
            y_rd[j].wait_recv()
            o = other + Qd + j * CK
            r1 = rc(o, xsA.at[j], xrA.at[j], x_nbr)
            r1.start()
            xA.append(r1)
            r2 = rc(o, zsA.at[j], zrA.at[j], z_nbr)
            r2.start()
            zA.append(r2)

        rwait(zrA.at[2])
        xB = rc(other + Qz + 2 * CK, xsB.at[0], xrB.at[0], x_nbr)
        xB.start()
        rwait(xrA.at[3])
        zB = rc(other + Qx + 3 * CK, zsB.at[0], zrB.at[0], z_nbr)
        zB.start()

        for k in range(KQ, KY):
            y_rd[k].wait_recv()
        for j in (0, 1, 2):
            rwait(xrA.at[j])
        for j in (0, 1, 3):
            rwait(zrA.at[j])
        rwait(xrB.at[0])
        rwait(zrB.at[0])

        for k in range(KY):
            y_rd[k].wait_send()
        for j in range(KQ):
            xA[j].wait_send()
            zA[j].wait_send()
        xB.wait_send()
        zB.wait_send()

    return pl.pallas_call(
        body,
        out_shape=jax.ShapeDtypeStruct((2 * m, n), jnp.bfloat16),
        in_specs=[pl.BlockSpec(memory_space=pltpu.VMEM)],
        out_specs=pl.BlockSpec(memory_space=pltpu.VMEM),
        scratch_shapes=[
            pltpu.SemaphoreType.DMA((KY,)),
            pltpu.SemaphoreType.DMA((KY,)),
            pltpu.SemaphoreType.DMA((KQ,)),
            pltpu.SemaphoreType.DMA((KQ,)),
            pltpu.SemaphoreType.DMA((KQ,)),
            pltpu.SemaphoreType.DMA((KQ,)),
            pltpu.SemaphoreType.DMA((1,)),
            pltpu.SemaphoreType.DMA((1,)),
            pltpu.SemaphoreType.DMA((1,)),
            pltpu.SemaphoreType.DMA((1,)),
        ],
        compiler_params=pltpu.CompilerParams(collective_id=0),
    )(x)


# device time: 19903 ns/iter; 1.0654x vs baseline; 1.0432x over previous
import jax
import jax.numpy as jnp
from jax import lax
from jax.experimental import pallas as pl
from jax.experimental.pallas import tpu as pltpu

CK = 64


def kernel(x):
    m, n = x.shape
    q = m // 4
    KQ = q // CK
    KB = KQ // 2
    KY = KQ + KB

    def body(x_ref, out_ref, ys, yr, xsA, xrA, zsA, zrA, xsB, xrB, zsB, zrB):
        my_x = lax.axis_index("x")
        my_y = lax.axis_index("y")
        my_z = lax.axis_index("z")
        qz = lax.rem(my_z, 2)
        y_nbr = (my_x, 1 - my_y, my_z)
        x_nbr = (1 - my_x, my_y, my_z)
        z_nbr = (my_x, my_y, my_z + 1 - 2 * qz)

        barrier = pltpu.get_barrier_semaphore()
        for nbr in (y_nbr, x_nbr, z_nbr):
            pl.semaphore_signal(
                barrier, inc=1, device_id=nbr,
                device_id_type=pl.DeviceIdType.MESH,
            )

        mine = my_y * m
        other = (1 - my_y) * m
        Qd = my_x * 2 * q + qz * q
        Qx = (1 - my_x) * 2 * q + qz * q
        Qz = my_x * 2 * q + (1 - qz) * q
        Qg = (1 - my_x) * 2 * q + (1 - qz) * q

        def cast(off, rows):
            out_ref[pl.ds(mine + off, rows), :] = x_ref[
                pl.ds(off, rows), :
            ].astype(jnp.bfloat16)

        def rc(off, ssem, rsem, dev):
            return pltpu.make_async_remote_copy(
                src_ref=out_ref.at[pl.ds(off, CK)],
                dst_ref=out_ref.at[pl.ds(off, CK)],
                send_sem=ssem,
                recv_sem=rsem,
                device_id=dev,
                device_id_type=pl.DeviceIdType.MESH,
            )

        def rwait(rsem):
            d = pltpu.make_async_remote_copy(
                src_ref=out_ref.at[pl.ds(other, CK)],
                dst_ref=out_ref.at[pl.ds(other, CK)],
                send_sem=ys.at[0],
                recv_sem=rsem,
                device_id=y_nbr,
                device_id_type=pl.DeviceIdType.MESH,
            )
            d.wait_recv()

        y_offs = [Qd + j * CK for j in range(KQ)] + [Qg + i * CK for i in range(KB)]
        for off in y_offs:
            cast(off, CK)

        pl.semaphore_wait(barrier, 3)

        y_rd = []
        for k, off in enumerate(y_offs):
            r = rc(mine + off, ys.at[k], yr.at[k], y_nbr)
            r.start()
            y_rd.append(r)

        cast(Qx, q)
        cast(Qz, q)
        cast(Qg + KB * CK, q - KB * CK)

        xA, zA = [], []
        for j in range(KQ):
            y_rd[j].wait_recv()
            o = other + Qd + j * CK
            r1 = rc(o, xsA.at[j], xrA.at[j], x_nbr)
            r1.start()
            xA.append(r1)
            r2 = rc(o, zsA.at[j], zrA.at[j], z_nbr)
            r2.start()
            zA.append(r2)

        xB_js = list(range(KQ // 2, 3 * KQ // 4))
        zB_js = list(range(3 * KQ // 4, KQ))
        xB, zB = [], []
        for i, jj in enumerate(xB_js):
            rwait(zrA.at[jj])
            r = rc(other + Qz + jj * CK, xsB.at[i], xrB.at[i], x_nbr)
            r.start()
            xB.append(r)
        for i, jj in enumerate(zB_js):
            rwait(xrA.at[jj])
            r = rc(other + Qx + jj * CK, zsB.at[i], zrB.at[i], z_nbr)
            r.start()
            zB.append(r)

        for k in range(KQ, KY):
            y_rd[k].wait_recv()
        for j in range(KQ):
            if j not in zB_js:
                rwait(xrA.at[j])
        for j in range(KQ):
            if j not in xB_js:
                rwait(zrA.at[j])
        for i in range(len(xB_js)):
            rwait(xrB.at[i])
        for i in range(len(zB_js)):
            rwait(zrB.at[i])

        for k in range(KY):
            y_rd[k].wait_send()
        for j in range(KQ):
            xA[j].wait_send()
            zA[j].wait_send()
        for r in xB + zB:
            r.wait_send()

    return pl.pallas_call(
        body,
        out_shape=jax.ShapeDtypeStruct((2 * m, n), jnp.bfloat16),
        in_specs=[pl.BlockSpec(memory_space=pltpu.VMEM)],
        out_specs=pl.BlockSpec(memory_space=pltpu.VMEM),
        scratch_shapes=[
            pltpu.SemaphoreType.DMA((KY,)),
            pltpu.SemaphoreType.DMA((KY,)),
            pltpu.SemaphoreType.DMA((KQ,)),
            pltpu.SemaphoreType.DMA((KQ,)),
            pltpu.SemaphoreType.DMA((KQ,)),
            pltpu.SemaphoreType.DMA((KQ,)),
            pltpu.SemaphoreType.DMA((max(KQ // 4, 1),)),
            pltpu.SemaphoreType.DMA((max(KQ // 4, 1),)),
            pltpu.SemaphoreType.DMA((max(KQ // 4, 1),)),
            pltpu.SemaphoreType.DMA((max(KQ // 4, 1),)),
        ],
        compiler_params=pltpu.CompilerParams(collective_id=0),
    )(x)


# device time: 19831 ns/iter; 1.0693x vs baseline; 1.0036x over previous
import jax
import jax.numpy as jnp
from jax import lax
from jax.experimental import pallas as pl
from jax.experimental.pallas import tpu as pltpu

CK = 32


def kernel(x):
    m, n = x.shape
    q = m // 4
    KQ = q // CK
    KB = KQ // 2
    KY = KQ + KB

    def body(x_ref, out_ref, ys, yr, xsA, xrA, zsA, zrA, xsB, xrB, zsB, zrB):
        my_x = lax.axis_index("x")
        my_y = lax.axis_index("y")
        my_z = lax.axis_index("z")
        qz = lax.rem(my_z, 2)
        y_nbr = (my_x, 1 - my_y, my_z)
        x_nbr = (1 - my_x, my_y, my_z)
        z_nbr = (my_x, my_y, my_z + 1 - 2 * qz)

        barrier = pltpu.get_barrier_semaphore()
        for nbr in (y_nbr, x_nbr, z_nbr):
            pl.semaphore_signal(
                barrier, inc=1, device_id=nbr,
                device_id_type=pl.DeviceIdType.MESH,
            )

        mine = my_y * m
        other = (1 - my_y) * m
        Qd = my_x * 2 * q + qz * q
        Qx = (1 - my_x) * 2 * q + qz * q
        Qz = my_x * 2 * q + (1 - qz) * q
        Qg = (1 - my_x) * 2 * q + (1 - qz) * q

        def cast(off, rows):
            out_ref[pl.ds(mine + off, rows), :] = x_ref[
                pl.ds(off, rows), :
            ].astype(jnp.bfloat16)

        def rc(off, ssem, rsem, dev):
            return pltpu.make_async_remote_copy(
                src_ref=out_ref.at[pl.ds(off, CK)],
                dst_ref=out_ref.at[pl.ds(off, CK)],
                send_sem=ssem,
                recv_sem=rsem,
                device_id=dev,
                device_id_type=pl.DeviceIdType.MESH,
            )

        def rwait(rsem):
            d = pltpu.make_async_remote_copy(
                src_ref=out_ref.at[pl.ds(other, CK)],
                dst_ref=out_ref.at[pl.ds(other, CK)],
                send_sem=ys.at[0],
                recv_sem=rsem,
                device_id=y_nbr,
                device_id_type=pl.DeviceIdType.MESH,
            )
            d.wait_recv()

        y_offs = [Qd + j * CK for j in range(KQ)] + [Qg + i * CK for i in range(KB)]
        for off in y_offs:
            cast(off, CK)

        pl.semaphore_wait(barrier, 3)

        y_rd = []
        for k, off in enumerate(y_offs):
            r = rc(mine + off, ys.at[k], yr.at[k], y_nbr)
            r.start()
            y_rd.append(r)

        cast(Qx, q)
        cast(Qz, q)
        cast(Qg + KB * CK, q - KB * CK)

        xA, zA = [], []
        for j in range(KQ):
            y_rd[j].wait_recv()
            o = other + Qd + j * CK
            r1 = rc(o, xsA.at[j], xrA.at[j], x_nbr)
            r1.start()
            xA.append(r1)
            r2 = rc(o, zsA.at[j], zrA.at[j], z_nbr)
            r2.start()
            zA.append(r2)

        xB_js = list(range(KQ // 2, 3 * KQ // 4))
        zB_js = list(range(3 * KQ // 4, KQ))
        xB, zB = [], []
        for i, jj in enumerate(xB_js):
            rwait(zrA.at[jj])
            r = rc(other + Qz + jj * CK, xsB.at[i], xrB.at[i], x_nbr)
            r.start()
            xB.append(r)
        for i, jj in enumerate(zB_js):
            rwait(xrA.at[jj])
            r = rc(other + Qx + jj * CK, zsB.at[i], zrB.at[i], z_nbr)
            r.start()
            zB.append(r)

        for k in range(KQ, KY):
            y_rd[k].wait_recv()
        for j in range(KQ):
            if j not in zB_js:
                rwait(xrA.at[j])
        for j in range(KQ):
            if j not in xB_js:
                rwait(zrA.at[j])
        for i in range(len(xB_js)):
            rwait(xrB.at[i])
        for i in range(len(zB_js)):
            rwait(zrB.at[i])

        for k in range(KY):
            y_rd[k].wait_send()
        for j in range(KQ):
            xA[j].wait_send()
            zA[j].wait_send()
        for r in xB + zB:
            r.wait_send()

    return pl.pallas_call(
        body,
        out_shape=jax.ShapeDtypeStruct((2 * m, n), jnp.bfloat16),
        in_specs=[pl.BlockSpec(memory_space=pltpu.VMEM)],
        out_specs=pl.BlockSpec(memory_space=pltpu.VMEM),
        scratch_shapes=[
            pltpu.SemaphoreType.DMA((KY,)),
            pltpu.SemaphoreType.DMA((KY,)),
            pltpu.SemaphoreType.DMA((KQ,)),
            pltpu.SemaphoreType.DMA((KQ,)),
            pltpu.SemaphoreType.DMA((KQ,)),
            pltpu.SemaphoreType.DMA((KQ,)),
            pltpu.SemaphoreType.DMA((max(KQ // 4, 1),)),
            pltpu.SemaphoreType.DMA((max(KQ // 4, 1),)),
            pltpu.SemaphoreType.DMA((max(KQ // 4, 1),)),
            pltpu.SemaphoreType.DMA((max(KQ // 4, 1),)),
        ],
        compiler_params=pltpu.CompilerParams(collective_id=0),
    )(x)


# device time: 19750 ns/iter; 1.0737x vs baseline; 1.0041x over previous
import jax
import jax.numpy as jnp
from jax import lax
from jax.experimental import pallas as pl
from jax.experimental.pallas import tpu as pltpu

CK = 32


def kernel(x):
    m, n = x.shape
    q = m // 4
    KQ = q // CK
    KB = KQ // 2
    KY = KQ + KB

    def body(x_ref, out_ref, ys, yr, xsA, xrA, zsA, zrA, xsB, xrB, zsB, zrB,
             ysync):
        my_x = lax.axis_index("x")
        my_y = lax.axis_index("y")
        my_z = lax.axis_index("z")
        qz = lax.rem(my_z, 2)
        y_nbr = (my_x, 1 - my_y, my_z)
        x_nbr = (1 - my_x, my_y, my_z)
        z_nbr = (my_x, my_y, my_z + 1 - 2 * qz)

        barrier = pltpu.get_barrier_semaphore()
        pl.semaphore_signal(
            ysync, inc=1, device_id=y_nbr,
            device_id_type=pl.DeviceIdType.MESH,
        )
        for nbr in (x_nbr, z_nbr):
            pl.semaphore_signal(
                barrier, inc=1, device_id=nbr,
                device_id_type=pl.DeviceIdType.MESH,
            )

        mine = my_y * m
        other = (1 - my_y) * m
        Qd = my_x * 2 * q + qz * q
        Qx = (1 - my_x) * 2 * q + qz * q
        Qz = my_x * 2 * q + (1 - qz) * q
        Qg = (1 - my_x) * 2 * q + (1 - qz) * q

        def cast(off, rows):
            out_ref[pl.ds(mine + off, rows), :] = x_ref[
                pl.ds(off, rows), :
            ].astype(jnp.bfloat16)

        def rc(off, ssem, rsem, dev):
            return pltpu.make_async_remote_copy(
                src_ref=out_ref.at[pl.ds(off, CK)],
                dst_ref=out_ref.at[pl.ds(off, CK)],
                send_sem=ssem,
                recv_sem=rsem,
                device_id=dev,
                device_id_type=pl.DeviceIdType.MESH,
            )

        def rwait(rsem):
            d = pltpu.make_async_remote_copy(
                src_ref=out_ref.at[pl.ds(other, CK)],
                dst_ref=out_ref.at[pl.ds(other, CK)],
                send_sem=ys.at[0],
                recv_sem=rsem,
                device_id=y_nbr,
                device_id_type=pl.DeviceIdType.MESH,
            )
            d.wait_recv()

        y_offs = [Qd + j * CK for j in range(KQ)] + [Qg + i * CK for i in range(KB)]
        for off in y_offs:
            cast(off, CK)

        pl.semaphore_wait(ysync, 1)

        y_rd = []
        for k, off in enumerate(y_offs):
            r = rc(mine + off, ys.at[k], yr.at[k], y_nbr)
            r.start()
            y_rd.append(r)

        cast(Qx, q)
        cast(Qz, q)
        cast(Qg + KB * CK, q - KB * CK)

        pl.semaphore_wait(barrier, 2)

        xA, zA = [], []
        for j in range(KQ):
            y_rd[j].wait_recv()
            o = other + Qd + j * CK
            r1 = rc(o, xsA.at[j], xrA.at[j], x_nbr)
            r1.start()
            xA.append(r1)
            r2 = rc(o, zsA.at[j], zrA.at[j], z_nbr)
            r2.start()
            zA.append(r2)

        xB_js = list(range(KQ // 2, 3 * KQ // 4))
        zB_js = list(range(3 * KQ // 4, KQ))
        xB, zB = [], []
        for i, jj in enumerate(xB_js):
            rwait(zrA.at[jj])
            r = rc(other + Qz + jj * CK, xsB.at[i], xrB.at[i], x_nbr)
            r.start()
            xB.append(r)
        for i, jj in enumerate(zB_js):
            rwait(xrA.at[jj])
            r = rc(other + Qx + jj * CK, zsB.at[i], zrB.at[i], z_nbr)
            r.start()
            zB.append(r)

        for k in range(KQ, KY):
            y_rd[k].wait_recv()
        for j in range(KQ):
            if j not in zB_js:
                rwait(xrA.at[j])
        for j in range(KQ):
            if j not in xB_js:
                rwait(zrA.at[j])
        for i in range(len(xB_js)):
            rwait(xrB.at[i])
        for i in range(len(zB_js)):
            rwait(zrB.at[i])

        for k in range(KY):
            y_rd[k].wait_send()
        for j in range(KQ):
            xA[j].wait_send()
            zA[j].wait_send()
        for r in xB + zB:
            r.wait_send()

    return pl.pallas_call(
        body,
        out_shape=jax.ShapeDtypeStruct((2 * m, n), jnp.bfloat16),
        in_specs=[pl.BlockSpec(memory_space=pltpu.VMEM)],
        out_specs=pl.BlockSpec(memory_space=pltpu.VMEM),
        scratch_shapes=[
            pltpu.SemaphoreType.DMA((KY,)),
            pltpu.SemaphoreType.DMA((KY,)),
            pltpu.SemaphoreType.DMA((KQ,)),
            pltpu.SemaphoreType.DMA((KQ,)),
            pltpu.SemaphoreType.DMA((KQ,)),
            pltpu.SemaphoreType.DMA((KQ,)),
            pltpu.SemaphoreType.DMA((max(KQ // 4, 1),)),
            pltpu.SemaphoreType.DMA((max(KQ // 4, 1),)),
            pltpu.SemaphoreType.DMA((max(KQ // 4, 1),)),
            pltpu.SemaphoreType.DMA((max(KQ // 4, 1),)),
            pltpu.SemaphoreType.REGULAR,
        ],
        compiler_params=pltpu.CompilerParams(collective_id=0),
    )(x)
